# baseline (speedup 1.0000x reference)
import functools

import jax
import jax.numpy as jnp
from jax import lax
from jax.experimental import pallas as pl
from jax.experimental.pallas import tpu as pltpu
from jax.experimental.pallas import tpu_sc as plsc

N_USER = 100000
N_ITEM = 1000000
EMBED_DIM = 32
BATCH = 16384

NC = 2
NS = 16
NW = NC * NS
R = BATCH // NW
CHUNK = 128
NCHUNK = R // CHUNK


def _bpr_body(u_hbm, i_hbm, j_hbm, eu_hbm, ei_hbm, out_ui_hbm, out_uj_hbm,
              uidx, iidx, jidx, ue, ie, je, oui, ouj, sem):
    wid = lax.axis_index("s") * NC + lax.axis_index("c")
    base = wid * R

    pltpu.sync_copy(u_hbm.at[wid], uidx)
    pltpu.sync_copy(i_hbm.at[wid], iidx)
    pltpu.sync_copy(j_hbm.at[wid], jidx)

    copies = []
    for c in range(NCHUNK):
        dst = pl.ds(c * CHUNK, CHUNK)
        copies.append(pltpu.async_copy(eu_hbm.at[uidx.at[c]], ue.at[dst], sem))
        copies.append(pltpu.async_copy(ei_hbm.at[iidx.at[c]], ie.at[dst], sem))
        copies.append(pltpu.async_copy(ei_hbm.at[jidx.at[c]], je.at[dst], sem))
    for cp in copies:
        cp.wait()

    lane = lax.iota(jnp.int32, 16)

    def group(g, _):
        rowv = lane + g * 16
        acc_ui = jnp.zeros((16,), jnp.float32)
        acc_uj = jnp.zeros((16,), jnp.float32)
        for d in range(EMBED_DIM):
            dv = jnp.full((16,), d, jnp.int32)
            uev = plsc.load_gather(ue, [rowv, dv])
            iev = plsc.load_gather(ie, [rowv, dv])
            jev = plsc.load_gather(je, [rowv, dv])
            acc_ui = acc_ui + uev * iev
            acc_uj = acc_uj + uev * jev
        oui[pl.ds(g * 16, 16)] = acc_ui
        ouj[pl.ds(g * 16, 16)] = acc_uj
        return _

    lax.fori_loop(0, R // 16, group, None)

    pltpu.sync_copy(oui, out_ui_hbm.at[pl.ds(base, R)])
    pltpu.sync_copy(ouj, out_uj_hbm.at[pl.ds(base, R)])


@jax.jit
def _bpr(u3, i3, j3, embed_user, embed_item):
    mesh = plsc.VectorSubcoreMesh(core_axis_name="c", subcore_axis_name="s")
    f = pl.kernel(
        _bpr_body,
        out_type=(
            jax.ShapeDtypeStruct((BATCH,), jnp.float32),
            jax.ShapeDtypeStruct((BATCH,), jnp.float32),
        ),
        mesh=mesh,
        compiler_params=pltpu.CompilerParams(
            needs_layout_passes=False, use_tc_tiling_on_sc=False),
        scratch_types=[
            pltpu.VMEM((NCHUNK, CHUNK), jnp.int32),
            pltpu.VMEM((NCHUNK, CHUNK), jnp.int32),
            pltpu.VMEM((NCHUNK, CHUNK), jnp.int32),
            pltpu.VMEM((R, EMBED_DIM), jnp.float32),
            pltpu.VMEM((R, EMBED_DIM), jnp.float32),
            pltpu.VMEM((R, EMBED_DIM), jnp.float32),
            pltpu.VMEM((R,), jnp.float32),
            pltpu.VMEM((R,), jnp.float32),
            pltpu.SemaphoreType.DMA,
        ],
    )
    return f(u3, i3, j3, embed_user, embed_item)


def kernel(u, i, j, embed_user, embed_item):
    u3 = u.astype(jnp.int32).reshape(NW, NCHUNK, CHUNK)
    i3 = i.astype(jnp.int32).reshape(NW, NCHUNK, CHUNK)
    j3 = j.astype(jnp.int32).reshape(NW, NCHUNK, CHUNK)
    p_ui, p_uj = _bpr(u3, i3, j3, embed_user, embed_item)
    return (p_ui.reshape(BATCH, 1), p_uj.reshape(BATCH, 1))
